# baseline (speedup 1.0000x reference)
import jax
import jax.numpy as jnp
from jax import lax
from jax.experimental import pallas as pl
from jax.experimental.pallas import tpu as pltpu
from jax.experimental.pallas import tpu_sc as plsc

N_FACTORS = 32
BATCH = 16384
NUM_CORES = 2
NUM_SUBCORES = 16
NW = NUM_CORES * NUM_SUBCORES
BPW = BATCH // NW
CHUNK = 128
NCH = BPW // CHUNK
LANES = 16
SUPER = 128
ROWS_PER_SUPER = SUPER // N_FACTORS


def _pmf_body(uidx_hbm, iidx_hbm, uemb_hbm, iemb_hbm, ub_hbm, ib_hbm,
              out_hbm, uidx_v, iidx_v, usup_v, isup_v,
              ubuf0, ubuf1, ibuf0, ibuf1, ubv, ibv, outv, sem0, sem1):
    wid = lax.axis_index("s") * NUM_CORES + lax.axis_index("c")
    base = wid * BPW

    for c in range(NCH):
        pltpu.sync_copy(uidx_hbm.at[pl.ds(base + c * CHUNK, CHUNK)],
                        uidx_v.at[c])
        pltpu.sync_copy(iidx_hbm.at[pl.ds(base + c * CHUNK, CHUNK)],
                        iidx_v.at[c])

    for c in range(NCH):
        for k in range(CHUNK // LANES):
            sl = pl.ds(k * LANES, LANES)
            usup_v[c, sl] = lax.shift_right_logical(uidx_v[c, sl], 2)
            isup_v[c, sl] = lax.shift_right_logical(iidx_v[c, sl], 2)

    ubufs = (ubuf0, ubuf1)
    ibufs = (ibuf0, ibuf1)
    sems = (sem0, sem1)

    def fire(c):
        s = sems[c % 2]
        bsl = pl.ds(c * CHUNK, CHUNK)
        return (
            pltpu.async_copy(uemb_hbm.at[usup_v.at[c]], ubufs[c % 2], s),
            pltpu.async_copy(iemb_hbm.at[isup_v.at[c]], ibufs[c % 2], s),
            pltpu.async_copy(ub_hbm.at[uidx_v.at[c]], ubv.at[bsl], s),
            pltpu.async_copy(ib_hbm.at[iidx_v.at[c]], ibv.at[bsl], s),
        )

    inflight = fire(0)
    for c in range(NCH):
        nxt = fire(c + 1) if c + 1 < NCH else None
        for cp in inflight:
            cp.wait()
        inflight = nxt

        ubuf = ubufs[c % 2]
        ibuf = ibufs[c % 2]

        def block(j, carry, c=c, ubuf=ubuf, ibuf=ibuf):
            b0 = c * CHUNK + j * LANES
            rows = j * LANES + lax.iota(jnp.int32, LANES)
            uix = uidx_v[c, pl.ds(j * LANES, LANES)]
            iix = iidx_v[c, pl.ds(j * LANES, LANES)]
            ucol = (uix & 3) * N_FACTORS
            icol = (iix & 3) * N_FACTORS
            acc = ubv[pl.ds(b0, LANES)] + ibv[pl.ds(b0, LANES)]
            for f in range(N_FACTORS):
                uv = plsc.load_gather(ubuf, [rows, ucol + f])
                iv = plsc.load_gather(ibuf, [rows, icol + f])
                acc = acc + uv * iv
            outv[pl.ds(b0, LANES)] = acc
            return carry

        lax.fori_loop(0, CHUNK // LANES, block, 0)

    pltpu.sync_copy(outv, out_hbm.at[pl.ds(base, BPW)])


def kernel(users_index, items_index, user_emb, item_emb, ub, ib):
    uemb2 = user_emb.reshape(-1, SUPER)
    iemb2 = item_emb.reshape(-1, SUPER)
    ubf = ub.reshape(-1)
    ibf = ib.reshape(-1)
    uidx = users_index.astype(jnp.int32)
    iidx = items_index.astype(jnp.int32)

    mesh = plsc.VectorSubcoreMesh(core_axis_name="c", subcore_axis_name="s")

    run = pl.kernel(
        _pmf_body,
        mesh=mesh,
        out_type=jax.ShapeDtypeStruct((BATCH,), jnp.float32),
        scratch_types=[
            pltpu.VMEM((NCH, CHUNK), jnp.int32),
            pltpu.VMEM((NCH, CHUNK), jnp.int32),
            pltpu.VMEM((NCH, CHUNK), jnp.int32),
            pltpu.VMEM((NCH, CHUNK), jnp.int32),
            pltpu.VMEM((CHUNK, SUPER), jnp.float32),
            pltpu.VMEM((CHUNK, SUPER), jnp.float32),
            pltpu.VMEM((CHUNK, SUPER), jnp.float32),
            pltpu.VMEM((CHUNK, SUPER), jnp.float32),
            pltpu.VMEM((BPW,), jnp.float32),
            pltpu.VMEM((BPW,), jnp.float32),
            pltpu.VMEM((BPW,), jnp.float32),
            pltpu.SemaphoreType.DMA,
            pltpu.SemaphoreType.DMA,
        ],
        compiler_params=pltpu.CompilerParams(needs_layout_passes=False),
    )
    return run(uidx, iidx, uemb2, iemb2, ubf, ibf)
